# Initial kernel scaffold; baseline (speedup 1.0000x reference)
import jax
import jax.numpy as jnp
from jax.experimental import pallas as pl


def kernel(x, edge_index, W1, b1, W2, b2):
    raise NotImplementedError("write your pallas kernel here")


# speedup vs baseline: 85.4896x; 85.4896x over previous
import functools

import jax
import jax.numpy as jnp
from jax import lax
from jax.experimental import pallas as pl
from jax.experimental.pallas import tpu as pltpu
from jax.experimental.pallas import tpu_sc as plsc

N_NODES_C = 100000
N_EDGES_C = 6400000

LANES = 128
ROWS = 800
NPAD = ROWS * LANES
PADNODE = NPAD - 1

NC = 2
NS = 16
NW = NC * NS
NPT = NPAD // NS

TBLK = 1600
EPAD = NW * TBLK * LANES
KB = 64
NITER = TBLK // KB

_mesh = plsc.VectorSubcoreMesh(core_axis_name="c", subcore_axis_name="s")


def _fill(buf, value, n):
    vec = jnp.full((16,), value, jnp.float32)

    def body(i, _):
        buf[pl.ds(i * 16, 16)] = vec
        return 0

    lax.fori_loop(0, n // 16, body, 0)


@functools.partial(
    pl.kernel,
    out_type=jax.ShapeDtypeStruct((NC * NPAD,), jnp.float32),
    mesh=_mesh,
    scratch_types=[
        pltpu.VMEM((KB, LANES), jnp.int32),
        pltpu.VMEM((LANES,), jnp.float32),
        pltpu.VMEM((NPT,), jnp.float32),
        pltpu.VMEM_SHARED((NPAD,), jnp.float32),
    ],
)
def _sc_degree(dst_hbm, out_hbm, dst_v, ones_v, node_v, acc_s):
    c = lax.axis_index("c")
    s = lax.axis_index("s")
    wid = c * NS + s
    _fill(ones_v, 1.0, LANES)
    _fill(node_v, 0.0, NPT)
    pltpu.sync_copy(node_v, acc_s.at[pl.ds(s * NPT, NPT)])
    plsc.subcore_barrier()

    def chunk(it, _):
        rowbase = wid * TBLK + it * KB
        pltpu.sync_copy(dst_hbm.at[pl.ds(rowbase, KB)], dst_v)

        def blk(j, _):
            pltpu.sync_copy(ones_v, acc_s.at[dst_v.at[j]], add=True)
            return 0

        lax.fori_loop(0, KB, blk, 0)
        return 0

    lax.fori_loop(0, NITER, chunk, 0)
    plsc.subcore_barrier()
    pltpu.sync_copy(acc_s.at[pl.ds(s * NPT, NPT)], node_v)
    pltpu.sync_copy(node_v, out_hbm.at[pl.ds(c * NPAD + s * NPT, NPT)])


@functools.partial(
    pl.kernel,
    out_type=jax.ShapeDtypeStruct((NC * 3 * NPAD,), jnp.float32),
    mesh=_mesh,
    scratch_types=[
        pltpu.VMEM((KB, LANES), jnp.int32),
        pltpu.VMEM((KB, LANES), jnp.int32),
        pltpu.VMEM((LANES,), jnp.float32),
        pltpu.VMEM((NPT,), jnp.float32),
        pltpu.VMEM_SHARED((NPAD,), jnp.float32),
        pltpu.VMEM_SHARED((NPAD,), jnp.float32),
        pltpu.VMEM_SHARED((NPAD,), jnp.float32),
        pltpu.VMEM_SHARED((NPAD,), jnp.float32),
        pltpu.VMEM_SHARED((NPAD,), jnp.float32),
        pltpu.VMEM_SHARED((NPAD,), jnp.float32),
    ],
)
def _sc_agg3(src_hbm, dst_hbm, g1_hbm, out_hbm, src_v, dst_v, val_v, node_v,
             t0, t1, t2, a0, a1, a2):
    c = lax.axis_index("c")
    s = lax.axis_index("s")
    wid = c * NS + s
    tabs = (t0, t1, t2)
    accs = (a0, a1, a2)
    sl = pl.ds(s * NPT, NPT)
    _fill(node_v, 0.0, NPT)
    for ch in range(3):
        pltpu.sync_copy(node_v, accs[ch].at[sl])
    for ch in range(3):
        pltpu.sync_copy(g1_hbm.at[pl.ds(ch * NPAD + s * NPT, NPT)], node_v)
        pltpu.sync_copy(node_v, tabs[ch].at[sl])
    plsc.subcore_barrier()

    def chunk(it, _):
        rowbase = wid * TBLK + it * KB
        pltpu.sync_copy(src_hbm.at[pl.ds(rowbase, KB)], src_v)
        pltpu.sync_copy(dst_hbm.at[pl.ds(rowbase, KB)], dst_v)

        def blk(j, _):
            for ch in range(3):
                pltpu.sync_copy(tabs[ch].at[src_v.at[j]], val_v)
                pltpu.sync_copy(val_v, accs[ch].at[dst_v.at[j]], add=True)
            return 0

        lax.fori_loop(0, KB, blk, 0)
        return 0

    lax.fori_loop(0, NITER, chunk, 0)
    plsc.subcore_barrier()
    for ch in range(3):
        pltpu.sync_copy(accs[ch].at[sl], node_v)
        pltpu.sync_copy(
            node_v, out_hbm.at[pl.ds((c * 3 + ch) * NPAD + s * NPT, NPT)])


@functools.partial(
    pl.kernel,
    out_type=jax.ShapeDtypeStruct((NC * NPAD,), jnp.float32),
    mesh=_mesh,
    scratch_types=[
        pltpu.VMEM((KB, LANES), jnp.int32),
        pltpu.VMEM((KB, LANES), jnp.int32),
        pltpu.VMEM((LANES,), jnp.float32),
        pltpu.VMEM((NPT,), jnp.float32),
        pltpu.VMEM_SHARED((NPAD,), jnp.float32),
        pltpu.VMEM_SHARED((NPAD,), jnp.float32),
    ],
)
def _sc_agg1(src_hbm, dst_hbm, g2_hbm, out_hbm, src_v, dst_v, val_v, node_v,
             tab_s, acc_s):
    c = lax.axis_index("c")
    s = lax.axis_index("s")
    wid = c * NS + s
    sl = pl.ds(s * NPT, NPT)
    _fill(node_v, 0.0, NPT)
    pltpu.sync_copy(node_v, acc_s.at[sl])
    pltpu.sync_copy(g2_hbm.at[pl.ds(s * NPT, NPT)], node_v)
    pltpu.sync_copy(node_v, tab_s.at[sl])
    plsc.subcore_barrier()

    def chunk(it, _):
        rowbase = wid * TBLK + it * KB
        pltpu.sync_copy(src_hbm.at[pl.ds(rowbase, KB)], src_v)
        pltpu.sync_copy(dst_hbm.at[pl.ds(rowbase, KB)], dst_v)

        def blk(j, _):
            pltpu.sync_copy(tab_s.at[src_v.at[j]], val_v)
            pltpu.sync_copy(val_v, acc_s.at[dst_v.at[j]], add=True)
            return 0

        lax.fori_loop(0, KB, blk, 0)
        return 0

    lax.fori_loop(0, NITER, chunk, 0)
    plsc.subcore_barrier()
    pltpu.sync_copy(acc_s.at[sl], node_v)
    pltpu.sync_copy(node_v, out_hbm.at[pl.ds(c * NPAD + s * NPT, NPT)])


def _tc_norm_body(degp_ref, xq_ref, dinv_ref, g1_ref):
    deg = degp_ref[0] + degp_ref[1] + 1.0
    dinv = lax.rsqrt(deg)
    dinv_ref[...] = dinv
    g1_ref[...] = xq_ref[...] * dinv[None]


def _tc_dense_body(aggp_ref, g1_ref, dinv_ref, w1_ref, b1_ref, w2_ref,
                   g2_ref):
    dinv = dinv_ref[...]
    xa = [(aggp_ref[0, ch] + aggp_ref[1, ch] + g1_ref[ch]) * dinv
          for ch in range(3)]
    s = jnp.zeros((ROWS, LANES), jnp.float32)
    for o in range(16):
        h = (xa[0] * w1_ref[0, o] + xa[1] * w1_ref[1, o]
             + xa[2] * w1_ref[2, o] + b1_ref[o])
        s = s + jnp.maximum(h, 0.0) * w2_ref[o, 0]
    g2_ref[...] = s * dinv


def _tc_final_body(a2p_ref, g2_ref, dinv_ref, b2_ref, out_ref):
    out_ref[...] = ((a2p_ref[0] + a2p_ref[1] + g2_ref[...])
                    * dinv_ref[...] + b2_ref[0])


def kernel(x, edge_index, W1, b1, W2, b2):
    f32 = jnp.float32
    ei = edge_index.astype(jnp.int32)
    npad_e = EPAD - N_EDGES_C
    pad_idx = jnp.full((npad_e,), PADNODE, jnp.int32)
    src2 = jnp.concatenate([ei[0], pad_idx]).reshape(EPAD // LANES, LANES)
    dst2 = jnp.concatenate([ei[1], pad_idx]).reshape(EPAD // LANES, LANES)

    xp = jnp.pad(x.astype(f32), ((0, NPAD - N_NODES_C), (0, 0)))
    xq = xp.T.reshape(3, ROWS, LANES)

    degp = _sc_degree(dst2)

    dinv2, g1 = pl.pallas_call(
        _tc_norm_body,
        out_shape=[
            jax.ShapeDtypeStruct((ROWS, LANES), f32),
            jax.ShapeDtypeStruct((3, ROWS, LANES), f32),
        ],
    )(degp.reshape(NC, ROWS, LANES), xq)

    aggp = _sc_agg3(src2, dst2, g1.reshape(3 * NPAD))

    g2 = pl.pallas_call(
        _tc_dense_body,
        in_specs=[
            pl.BlockSpec(memory_space=pltpu.VMEM),
            pl.BlockSpec(memory_space=pltpu.VMEM),
            pl.BlockSpec(memory_space=pltpu.VMEM),
            pl.BlockSpec(memory_space=pltpu.SMEM),
            pl.BlockSpec(memory_space=pltpu.SMEM),
            pl.BlockSpec(memory_space=pltpu.SMEM),
        ],
        out_shape=jax.ShapeDtypeStruct((ROWS, LANES), f32),
    )(aggp.reshape(NC, 3, ROWS, LANES), g1, dinv2,
      W1.astype(f32), b1.astype(f32), W2.astype(f32))

    a2p = _sc_agg1(src2, dst2, g2.reshape(NPAD))

    out2 = pl.pallas_call(
        _tc_final_body,
        in_specs=[
            pl.BlockSpec(memory_space=pltpu.VMEM),
            pl.BlockSpec(memory_space=pltpu.VMEM),
            pl.BlockSpec(memory_space=pltpu.VMEM),
            pl.BlockSpec(memory_space=pltpu.SMEM),
        ],
        out_shape=jax.ShapeDtypeStruct((ROWS, LANES), f32),
    )(a2p.reshape(NC, ROWS, LANES), g2, dinv2, b2.astype(f32))

    return out2.reshape(NPAD)[:N_NODES_C]


# speedup vs baseline: 108.1283x; 1.2648x over previous
import functools

import jax
import jax.numpy as jnp
from jax import lax
from jax.experimental import pallas as pl
from jax.experimental.pallas import tpu as pltpu
from jax.experimental.pallas import tpu_sc as plsc

N_NODES_C = 100000
N_EDGES_C = 6400000

LANES = 128
ROWS = 800
NPAD = ROWS * LANES
PADNODE = NPAD - 1

NC = 2
NS = 16
NW = NC * NS
NPT = NPAD // NS

TBLK = 1600
EPAD = NW * TBLK * LANES
KB = 64
CHUNKE = KB * LANES
NITER = TBLK // KB

_mesh = plsc.VectorSubcoreMesh(core_axis_name="c", subcore_axis_name="s")


def _fill(buf, value, n):
    vec = jnp.full((16,), value, jnp.float32)

    def body(i, _):
        buf[pl.ds(i * 16, 16)] = vec
        return 0

    lax.fori_loop(0, n // 16, body, 0)


@functools.partial(
    pl.kernel,
    out_type=jax.ShapeDtypeStruct((NC * NPAD,), jnp.float32),
    mesh=_mesh,
    scratch_types=[
        pltpu.VMEM((KB, LANES), jnp.int32),
        pltpu.VMEM((LANES,), jnp.float32),
        pltpu.VMEM((NPT,), jnp.float32),
        pltpu.VMEM_SHARED((NPAD,), jnp.float32),
        pltpu.SemaphoreType.DMA,
    ],
)
def _sc_degree(dst_hbm, out_hbm, dst_v, ones_v, node_v, acc_s, sem_s):
    c = lax.axis_index("c")
    s = lax.axis_index("s")
    wid = c * NS + s
    _fill(ones_v, 1.0, LANES)
    _fill(node_v, 0.0, NPT)
    pltpu.sync_copy(node_v, acc_s.at[pl.ds(s * NPT, NPT)])
    plsc.subcore_barrier()

    def chunk(it, _):
        rowbase = wid * TBLK + it * KB
        pltpu.sync_copy(dst_hbm.at[pl.ds(rowbase, KB)], dst_v)

        def fire(j, _):
            pltpu.async_copy(ones_v, acc_s.at[dst_v.at[j]], sem_s, add=True)
            return 0

        lax.fori_loop(0, KB, fire, 0)

        def drain(j, _):
            pltpu.make_async_copy(ones_v, acc_s.at[dst_v.at[0]], sem_s).wait()
            return 0

        lax.fori_loop(0, KB, drain, 0)
        return 0

    lax.fori_loop(0, NITER, chunk, 0)
    plsc.subcore_barrier()
    pltpu.sync_copy(acc_s.at[pl.ds(s * NPT, NPT)], node_v)
    pltpu.sync_copy(node_v, out_hbm.at[pl.ds(c * NPAD + s * NPT, NPT)])


@functools.partial(
    pl.kernel,
    out_type=jax.ShapeDtypeStruct((NC * 3 * NPAD,), jnp.float32),
    mesh=_mesh,
    scratch_types=[
        pltpu.VMEM((CHUNKE,), jnp.int32),
        pltpu.VMEM((KB, LANES), jnp.int32),
        pltpu.VMEM((CHUNKE,), jnp.float32),
        pltpu.VMEM((CHUNKE,), jnp.float32),
        pltpu.VMEM((CHUNKE,), jnp.float32),
        pltpu.VMEM((NPT,), jnp.float32),
        pltpu.VMEM_SHARED((NPAD,), jnp.float32),
        pltpu.VMEM_SHARED((NPAD,), jnp.float32),
        pltpu.VMEM_SHARED((NPAD,), jnp.float32),
        pltpu.VMEM_SHARED((NPAD,), jnp.float32),
        pltpu.VMEM_SHARED((NPAD,), jnp.float32),
        pltpu.VMEM_SHARED((NPAD,), jnp.float32),
        pltpu.SemaphoreType.DMA,
        pltpu.SemaphoreType.DMA,
    ],
)
def _sc_agg3(src_hbm, dst_hbm, g1_hbm, out_hbm, src_v, dst_v, gb0, gb1, gb2,
             node_v, t0, t1, t2, a0, a1, a2, sem_g, sem_s):
    c = lax.axis_index("c")
    s = lax.axis_index("s")
    wid = c * NS + s
    tabs = (t0, t1, t2)
    accs = (a0, a1, a2)
    gbs = (gb0, gb1, gb2)
    sl = pl.ds(s * NPT, NPT)
    _fill(node_v, 0.0, NPT)
    for ch in range(3):
        pltpu.sync_copy(node_v, accs[ch].at[sl])
    for ch in range(3):
        pltpu.sync_copy(g1_hbm.at[pl.ds(ch * NPAD + s * NPT, NPT)], node_v)
        pltpu.sync_copy(node_v, tabs[ch].at[sl])
    plsc.subcore_barrier()

    def chunk(it, _):
        rowbase = wid * TBLK + it * KB
        pltpu.sync_copy(src_hbm.at[pl.ds(rowbase * LANES, CHUNKE)], src_v)
        pltpu.sync_copy(dst_hbm.at[pl.ds(rowbase, KB)], dst_v)
        descs = [pltpu.async_copy(tabs[ch].at[src_v], gbs[ch], sem_g)
                 for ch in range(3)]
        for d in descs:
            d.wait()

        def fire(j, _):
            row = pl.ds(pl.multiple_of(j * LANES, LANES), LANES)
            for ch in range(3):
                pltpu.async_copy(gbs[ch].at[row], accs[ch].at[dst_v.at[j]],
                                 sem_s, add=True)
            return 0

        lax.fori_loop(0, KB, fire, 0)

        def drain(j, _):
            for ch in range(3):
                pltpu.make_async_copy(gbs[ch].at[pl.ds(0, LANES)],
                                      accs[ch].at[dst_v.at[0]], sem_s).wait()
            return 0

        lax.fori_loop(0, KB, drain, 0)
        return 0

    lax.fori_loop(0, NITER, chunk, 0)
    plsc.subcore_barrier()
    for ch in range(3):
        pltpu.sync_copy(accs[ch].at[sl], node_v)
        pltpu.sync_copy(
            node_v, out_hbm.at[pl.ds((c * 3 + ch) * NPAD + s * NPT, NPT)])


@functools.partial(
    pl.kernel,
    out_type=jax.ShapeDtypeStruct((NC * NPAD,), jnp.float32),
    mesh=_mesh,
    scratch_types=[
        pltpu.VMEM((CHUNKE,), jnp.int32),
        pltpu.VMEM((KB, LANES), jnp.int32),
        pltpu.VMEM((CHUNKE,), jnp.float32),
        pltpu.VMEM((NPT,), jnp.float32),
        pltpu.VMEM_SHARED((NPAD,), jnp.float32),
        pltpu.VMEM_SHARED((NPAD,), jnp.float32),
        pltpu.SemaphoreType.DMA,
        pltpu.SemaphoreType.DMA,
    ],
)
def _sc_agg1(src_hbm, dst_hbm, g2_hbm, out_hbm, src_v, dst_v, gb0, node_v,
             tab_s, acc_s, sem_g, sem_s):
    c = lax.axis_index("c")
    s = lax.axis_index("s")
    wid = c * NS + s
    sl = pl.ds(s * NPT, NPT)
    _fill(node_v, 0.0, NPT)
    pltpu.sync_copy(node_v, acc_s.at[sl])
    pltpu.sync_copy(g2_hbm.at[pl.ds(s * NPT, NPT)], node_v)
    pltpu.sync_copy(node_v, tab_s.at[sl])
    plsc.subcore_barrier()

    def chunk(it, _):
        rowbase = wid * TBLK + it * KB
        pltpu.sync_copy(src_hbm.at[pl.ds(rowbase * LANES, CHUNKE)], src_v)
        pltpu.sync_copy(dst_hbm.at[pl.ds(rowbase, KB)], dst_v)
        pltpu.async_copy(tab_s.at[src_v], gb0, sem_g).wait()

        def fire(j, _):
            row = pl.ds(pl.multiple_of(j * LANES, LANES), LANES)
            pltpu.async_copy(gb0.at[row],
                             acc_s.at[dst_v.at[j]], sem_s, add=True)
            return 0

        lax.fori_loop(0, KB, fire, 0)

        def drain(j, _):
            pltpu.make_async_copy(gb0.at[pl.ds(0, LANES)],
                                  acc_s.at[dst_v.at[0]], sem_s).wait()
            return 0

        lax.fori_loop(0, KB, drain, 0)
        return 0

    lax.fori_loop(0, NITER, chunk, 0)
    plsc.subcore_barrier()
    pltpu.sync_copy(acc_s.at[sl], node_v)
    pltpu.sync_copy(node_v, out_hbm.at[pl.ds(c * NPAD + s * NPT, NPT)])


def _tc_norm_body(degp_ref, xq_ref, dinv_ref, g1_ref):
    deg = degp_ref[0] + degp_ref[1] + 1.0
    dinv = lax.rsqrt(deg)
    dinv_ref[...] = dinv
    g1_ref[...] = xq_ref[...] * dinv[None]


def _tc_dense_body(aggp_ref, g1_ref, dinv_ref, w1_ref, b1_ref, w2_ref,
                   g2_ref):
    dinv = dinv_ref[...]
    xa = [(aggp_ref[0, ch] + aggp_ref[1, ch] + g1_ref[ch]) * dinv
          for ch in range(3)]
    s = jnp.zeros((ROWS, LANES), jnp.float32)
    for o in range(16):
        h = (xa[0] * w1_ref[0, o] + xa[1] * w1_ref[1, o]
             + xa[2] * w1_ref[2, o] + b1_ref[o])
        s = s + jnp.maximum(h, 0.0) * w2_ref[o, 0]
    g2_ref[...] = s * dinv


def _tc_final_body(a2p_ref, g2_ref, dinv_ref, b2_ref, out_ref):
    out_ref[...] = ((a2p_ref[0] + a2p_ref[1] + g2_ref[...])
                    * dinv_ref[...] + b2_ref[0])


def kernel(x, edge_index, W1, b1, W2, b2):
    f32 = jnp.float32
    ei = edge_index.astype(jnp.int32)
    npad_e = EPAD - N_EDGES_C
    pad_idx = jnp.full((npad_e,), PADNODE, jnp.int32)
    src1 = jnp.concatenate([ei[0], pad_idx])
    dst2 = jnp.concatenate([ei[1], pad_idx]).reshape(EPAD // LANES, LANES)

    xp = jnp.pad(x.astype(f32), ((0, NPAD - N_NODES_C), (0, 0)))
    xq = xp.T.reshape(3, ROWS, LANES)

    degp = _sc_degree(dst2)

    dinv2, g1 = pl.pallas_call(
        _tc_norm_body,
        out_shape=[
            jax.ShapeDtypeStruct((ROWS, LANES), f32),
            jax.ShapeDtypeStruct((3, ROWS, LANES), f32),
        ],
    )(degp.reshape(NC, ROWS, LANES), xq)

    aggp = _sc_agg3(src1, dst2, g1.reshape(3 * NPAD))

    g2 = pl.pallas_call(
        _tc_dense_body,
        in_specs=[
            pl.BlockSpec(memory_space=pltpu.VMEM),
            pl.BlockSpec(memory_space=pltpu.VMEM),
            pl.BlockSpec(memory_space=pltpu.VMEM),
            pl.BlockSpec(memory_space=pltpu.SMEM),
            pl.BlockSpec(memory_space=pltpu.SMEM),
            pl.BlockSpec(memory_space=pltpu.SMEM),
        ],
        out_shape=jax.ShapeDtypeStruct((ROWS, LANES), f32),
    )(aggp.reshape(NC, 3, ROWS, LANES), g1, dinv2,
      W1.astype(f32), b1.astype(f32), W2.astype(f32))

    a2p = _sc_agg1(src1, dst2, g2.reshape(NPAD))

    out2 = pl.pallas_call(
        _tc_final_body,
        in_specs=[
            pl.BlockSpec(memory_space=pltpu.VMEM),
            pl.BlockSpec(memory_space=pltpu.VMEM),
            pl.BlockSpec(memory_space=pltpu.VMEM),
            pl.BlockSpec(memory_space=pltpu.SMEM),
        ],
        out_shape=jax.ShapeDtypeStruct((ROWS, LANES), f32),
    )(a2p.reshape(NC, ROWS, LANES), g2, dinv2, b2.astype(f32))

    return out2.reshape(NPAD)[:N_NODES_C]


# speedup vs baseline: 236.5543x; 2.1877x over previous
import functools

import jax
import jax.numpy as jnp
from jax import lax
from jax.experimental import pallas as pl
from jax.experimental.pallas import tpu as pltpu
from jax.experimental.pallas import tpu_sc as plsc

N_NODES_C = 100000
N_EDGES_C = 6400000

LANES = 128
ROWS = 800
NPAD = ROWS * LANES
PADNODE = NPAD - 1

NC = 2
NS = 16
NW = NC * NS
NPT = NPAD // NS

TBLK = 1600
EPAD = NW * TBLK * LANES
KB = 64
CHUNKE = KB * LANES
NITER = TBLK // KB

_mesh = plsc.VectorSubcoreMesh(core_axis_name="c", subcore_axis_name="s")


def _fill(buf, value, n):
    vec = jnp.full((16,), value, jnp.float32)

    def body(i, _):
        buf[pl.ds(i * 16, 16)] = vec
        return 0

    lax.fori_loop(0, n // 16, body, 0)


@functools.partial(
    pl.kernel,
    out_type=jax.ShapeDtypeStruct((NC * NPAD,), jnp.float32),
    mesh=_mesh,
    scratch_types=[
        pltpu.VMEM((KB, LANES), jnp.int32),
        pltpu.VMEM((LANES,), jnp.float32),
        pltpu.VMEM((NPT,), jnp.float32),
        pltpu.VMEM_SHARED((NPAD,), jnp.float32),
        pltpu.SemaphoreType.DMA,
    ],
)
def _sc_degree(dst_hbm, out_hbm, dst_v, ones_v, node_v, acc_s, sem_s):
    c = lax.axis_index("c")
    s = lax.axis_index("s")
    wid = c * NS + s
    _fill(ones_v, 1.0, LANES)
    _fill(node_v, 0.0, NPT)
    pltpu.sync_copy(node_v, acc_s.at[pl.ds(s * NPT, NPT)])
    plsc.subcore_barrier()

    def chunk(it, _):
        rowbase = wid * TBLK + it * KB
        pltpu.sync_copy(dst_hbm.at[pl.ds(rowbase, KB)], dst_v)

        def fire(j, _):
            pltpu.async_copy(ones_v, acc_s.at[dst_v.at[j]], sem_s, add=True)
            return 0

        lax.fori_loop(0, KB, fire, 0)

        def drain(j, _):
            pltpu.make_async_copy(ones_v, acc_s.at[dst_v.at[0]], sem_s).wait()
            return 0

        lax.fori_loop(0, KB, drain, 0)
        return 0

    lax.fori_loop(0, NITER, chunk, 0)
    plsc.subcore_barrier()
    pltpu.sync_copy(acc_s.at[pl.ds(s * NPT, NPT)], node_v)
    pltpu.sync_copy(node_v, out_hbm.at[pl.ds(c * NPAD + s * NPT, NPT)])


@functools.partial(
    pl.kernel,
    out_type=jax.ShapeDtypeStruct((NC * 3 * NPAD,), jnp.float32),
    mesh=_mesh,
    scratch_types=[
        pltpu.VMEM((CHUNKE,), jnp.int32),
        pltpu.VMEM((KB, LANES), jnp.int32),
        pltpu.VMEM((CHUNKE,), jnp.float32),
        pltpu.VMEM((CHUNKE,), jnp.float32),
        pltpu.VMEM((CHUNKE,), jnp.float32),
        pltpu.VMEM((NPT,), jnp.float32),
        pltpu.VMEM_SHARED((NPAD,), jnp.float32),
        pltpu.VMEM_SHARED((NPAD,), jnp.float32),
        pltpu.VMEM_SHARED((NPAD,), jnp.float32),
        pltpu.VMEM_SHARED((NPAD,), jnp.float32),
        pltpu.VMEM_SHARED((NPAD,), jnp.float32),
        pltpu.VMEM_SHARED((NPAD,), jnp.float32),
        pltpu.SemaphoreType.DMA,
        pltpu.SemaphoreType.DMA,
    ],
)
def _sc_agg3(src_hbm, dst_hbm, g1_hbm, out_hbm, src_v, dst_v, gb0, gb1, gb2,
             node_v, t0, t1, t2, a0, a1, a2, sem_g, sem_s):
    c = lax.axis_index("c")
    s = lax.axis_index("s")
    wid = c * NS + s
    tabs = (t0, t1, t2)
    accs = (a0, a1, a2)
    gbs = (gb0, gb1, gb2)
    sl = pl.ds(s * NPT, NPT)
    _fill(node_v, 0.0, NPT)
    for ch in range(3):
        pltpu.sync_copy(node_v, accs[ch].at[sl])
    for ch in range(3):
        pltpu.sync_copy(g1_hbm.at[pl.ds(ch * NPAD + s * NPT, NPT)], node_v)
        pltpu.sync_copy(node_v, tabs[ch].at[sl])
    plsc.subcore_barrier()

    def chunk(it, _):
        rowbase = wid * TBLK + it * KB
        pltpu.sync_copy(src_hbm.at[pl.ds(rowbase * LANES, CHUNKE)], src_v)
        pltpu.sync_copy(dst_hbm.at[pl.ds(rowbase, KB)], dst_v)
        descs = [pltpu.async_copy(tabs[ch].at[src_v], gbs[ch], sem_g)
                 for ch in range(3)]
        for d in descs:
            d.wait()

        def fire(j, _):
            row = pl.ds(pl.multiple_of(j * LANES, LANES), LANES)
            for ch in range(3):
                pltpu.async_copy(gbs[ch].at[row], accs[ch].at[dst_v.at[j]],
                                 sem_s, add=True)
            return 0

        lax.fori_loop(0, KB, fire, 0)

        def drain(j, _):
            for ch in range(3):
                pltpu.make_async_copy(gbs[ch].at[pl.ds(0, LANES)],
                                      accs[ch].at[dst_v.at[0]], sem_s).wait()
            return 0

        lax.fori_loop(0, KB, drain, 0)
        return 0

    lax.fori_loop(0, NITER, chunk, 0)
    plsc.subcore_barrier()
    for ch in range(3):
        pltpu.sync_copy(accs[ch].at[sl], node_v)
        pltpu.sync_copy(
            node_v, out_hbm.at[pl.ds((c * 3 + ch) * NPAD + s * NPT, NPT)])


@functools.partial(
    pl.kernel,
    out_type=jax.ShapeDtypeStruct((NC * NPAD,), jnp.float32),
    mesh=_mesh,
    scratch_types=[
        pltpu.VMEM((CHUNKE,), jnp.int32),
        pltpu.VMEM((KB, LANES), jnp.int32),
        pltpu.VMEM((CHUNKE,), jnp.float32),
        pltpu.VMEM((NPT,), jnp.float32),
        pltpu.VMEM_SHARED((NPAD,), jnp.float32),
        pltpu.VMEM_SHARED((NPAD,), jnp.float32),
        pltpu.SemaphoreType.DMA,
        pltpu.SemaphoreType.DMA,
    ],
)
def _sc_agg1(src_hbm, dst_hbm, g2_hbm, out_hbm, src_v, dst_v, gb0, node_v,
             tab_s, acc_s, sem_g, sem_s):
    c = lax.axis_index("c")
    s = lax.axis_index("s")
    wid = c * NS + s
    sl = pl.ds(s * NPT, NPT)
    _fill(node_v, 0.0, NPT)
    pltpu.sync_copy(node_v, acc_s.at[sl])
    pltpu.sync_copy(g2_hbm.at[pl.ds(s * NPT, NPT)], node_v)
    pltpu.sync_copy(node_v, tab_s.at[sl])
    plsc.subcore_barrier()

    def chunk(it, _):
        rowbase = wid * TBLK + it * KB
        pltpu.sync_copy(src_hbm.at[pl.ds(rowbase * LANES, CHUNKE)], src_v)
        pltpu.sync_copy(dst_hbm.at[pl.ds(rowbase, KB)], dst_v)
        pltpu.async_copy(tab_s.at[src_v], gb0, sem_g).wait()

        def fire(j, _):
            row = pl.ds(pl.multiple_of(j * LANES, LANES), LANES)
            pltpu.async_copy(gb0.at[row],
                             acc_s.at[dst_v.at[j]], sem_s, add=True)
            return 0

        lax.fori_loop(0, KB, fire, 0)

        def drain(j, _):
            pltpu.make_async_copy(gb0.at[pl.ds(0, LANES)],
                                  acc_s.at[dst_v.at[0]], sem_s).wait()
            return 0

        lax.fori_loop(0, KB, drain, 0)
        return 0

    lax.fori_loop(0, NITER, chunk, 0)
    plsc.subcore_barrier()
    pltpu.sync_copy(acc_s.at[sl], node_v)
    pltpu.sync_copy(node_v, out_hbm.at[pl.ds(c * NPAD + s * NPT, NPT)])


def _tc_norm_body(degp_ref, xq_ref, dinv_ref, g1_ref):
    deg = degp_ref[0] + degp_ref[1] + 1.0
    dinv = lax.rsqrt(deg)
    dinv_ref[...] = dinv
    g1_ref[...] = xq_ref[...] * dinv[None]


def _tc_dense_body(aggp_ref, g1_ref, dinv_ref, w1_ref, b1_ref, w2_ref,
                   g2_ref):
    dinv = dinv_ref[...]
    xa = [(aggp_ref[0, ch] + aggp_ref[1, ch] + g1_ref[ch]) * dinv
          for ch in range(3)]
    s = jnp.zeros((ROWS, LANES), jnp.float32)
    for o in range(16):
        h = (xa[0] * w1_ref[0, o] + xa[1] * w1_ref[1, o]
             + xa[2] * w1_ref[2, o] + b1_ref[o])
        s = s + jnp.maximum(h, 0.0) * w2_ref[o, 0]
    g2_ref[...] = s * dinv


def _tc_final_body(a2p_ref, g2_ref, dinv_ref, b2_ref, out_ref):
    out_ref[...] = ((a2p_ref[0] + a2p_ref[1] + g2_ref[...])
                    * dinv_ref[...] + b2_ref[0])


def kernel(x, edge_index, W1, b1, W2, b2):
    f32 = jnp.float32
    ei = edge_index.astype(jnp.int32)
    npad_e = EPAD - N_EDGES_C
    pad_idx = (N_NODES_C
               + jnp.arange(npad_e, dtype=jnp.int32) % (NPAD - N_NODES_C))
    src1 = jnp.concatenate([ei[0], pad_idx])
    dst2 = jnp.concatenate([ei[1], pad_idx]).reshape(EPAD // LANES, LANES)

    xp = jnp.pad(x.astype(f32), ((0, NPAD - N_NODES_C), (0, 0)))
    xq = xp.T.reshape(3, ROWS, LANES)

    degp = _sc_degree(dst2)

    dinv2, g1 = pl.pallas_call(
        _tc_norm_body,
        out_shape=[
            jax.ShapeDtypeStruct((ROWS, LANES), f32),
            jax.ShapeDtypeStruct((3, ROWS, LANES), f32),
        ],
    )(degp.reshape(NC, ROWS, LANES), xq)

    aggp = _sc_agg3(src1, dst2, g1.reshape(3 * NPAD))

    g2 = pl.pallas_call(
        _tc_dense_body,
        in_specs=[
            pl.BlockSpec(memory_space=pltpu.VMEM),
            pl.BlockSpec(memory_space=pltpu.VMEM),
            pl.BlockSpec(memory_space=pltpu.VMEM),
            pl.BlockSpec(memory_space=pltpu.SMEM),
            pl.BlockSpec(memory_space=pltpu.SMEM),
            pl.BlockSpec(memory_space=pltpu.SMEM),
        ],
        out_shape=jax.ShapeDtypeStruct((ROWS, LANES), f32),
    )(aggp.reshape(NC, 3, ROWS, LANES), g1, dinv2,
      W1.astype(f32), b1.astype(f32), W2.astype(f32))

    a2p = _sc_agg1(src1, dst2, g2.reshape(NPAD))

    out2 = pl.pallas_call(
        _tc_final_body,
        in_specs=[
            pl.BlockSpec(memory_space=pltpu.VMEM),
            pl.BlockSpec(memory_space=pltpu.VMEM),
            pl.BlockSpec(memory_space=pltpu.VMEM),
            pl.BlockSpec(memory_space=pltpu.SMEM),
        ],
        out_shape=jax.ShapeDtypeStruct((ROWS, LANES), f32),
    )(a2p.reshape(NC, ROWS, LANES), g2, dinv2, b2.astype(f32))

    return out2.reshape(NPAD)[:N_NODES_C]


# speedup vs baseline: 236.6456x; 1.0004x over previous
import functools

import jax
import jax.numpy as jnp
from jax import lax
from jax.experimental import pallas as pl
from jax.experimental.pallas import tpu as pltpu
from jax.experimental.pallas import tpu_sc as plsc

N_NODES_C = 100000
N_EDGES_C = 6400000

LANES = 128
ROWS = 800
NPAD = ROWS * LANES
PADNODE = NPAD - 1

NC = 2
NS = 16
NW = NC * NS
NPT = NPAD // NS

TBLK = 1600
EPAD = NW * TBLK * LANES
KB = 64
CHUNKE = KB * LANES
NITER = TBLK // KB

_mesh = plsc.VectorSubcoreMesh(core_axis_name="c", subcore_axis_name="s")


def _fill(buf, value, n):
    vec = jnp.full((16,), value, jnp.float32)

    def body(i, _):
        buf[pl.ds(i * 16, 16)] = vec
        return 0

    lax.fori_loop(0, n // 16, body, 0)


@functools.partial(
    pl.kernel,
    out_type=jax.ShapeDtypeStruct((NC * NPAD,), jnp.float32),
    mesh=_mesh,
    scratch_types=[
        pltpu.VMEM((KB, LANES), jnp.int32),
        pltpu.VMEM((LANES,), jnp.float32),
        pltpu.VMEM((NPT,), jnp.float32),
        pltpu.VMEM_SHARED((NPAD,), jnp.float32),
        pltpu.SemaphoreType.DMA,
    ],
)
def _sc_degree(dst_hbm, out_hbm, dst_v, ones_v, node_v, acc_s, sem_s):
    c = lax.axis_index("c")
    s = lax.axis_index("s")
    wid = c * NS + s
    _fill(ones_v, 1.0, LANES)
    _fill(node_v, 0.0, NPT)
    pltpu.sync_copy(node_v, acc_s.at[pl.ds(s * NPT, NPT)])
    plsc.subcore_barrier()

    def chunk(it, _):
        rowbase = wid * TBLK + it * KB
        pltpu.sync_copy(dst_hbm.at[pl.ds(rowbase, KB)], dst_v)

        def fire(j, _):
            pltpu.async_copy(ones_v, acc_s.at[dst_v.at[j]], sem_s, add=True)
            return 0

        lax.fori_loop(0, KB, fire, 0)

        def drain(j, _):
            pltpu.make_async_copy(ones_v, acc_s.at[dst_v.at[0]], sem_s).wait()
            return 0

        lax.fori_loop(0, KB, drain, 0)
        return 0

    lax.fori_loop(0, NITER, chunk, 0)
    plsc.subcore_barrier()
    pltpu.sync_copy(acc_s.at[pl.ds(s * NPT, NPT)], node_v)
    pltpu.sync_copy(node_v, out_hbm.at[pl.ds(c * NPAD + s * NPT, NPT)])


@functools.partial(
    pl.kernel,
    out_type=jax.ShapeDtypeStruct((NC * 3 * NPAD,), jnp.float32),
    mesh=_mesh,
    scratch_types=[
        pltpu.VMEM((CHUNKE,), jnp.int32),
        pltpu.VMEM((KB, LANES), jnp.int32),
        pltpu.VMEM((CHUNKE,), jnp.float32),
        pltpu.VMEM((CHUNKE,), jnp.float32),
        pltpu.VMEM((CHUNKE,), jnp.float32),
        pltpu.VMEM((NPT,), jnp.float32),
        pltpu.VMEM_SHARED((NPAD,), jnp.float32),
        pltpu.VMEM_SHARED((NPAD,), jnp.float32),
        pltpu.VMEM_SHARED((NPAD,), jnp.float32),
        pltpu.VMEM_SHARED((NPAD,), jnp.float32),
        pltpu.VMEM_SHARED((NPAD,), jnp.float32),
        pltpu.VMEM_SHARED((NPAD,), jnp.float32),
        pltpu.SemaphoreType.DMA,
        pltpu.SemaphoreType.DMA,
    ],
)
def _sc_agg3(src_hbm, dst_hbm, g1_hbm, out_hbm, src_v, dst_v, gb0, gb1, gb2,
             node_v, t0, t1, t2, a0, a1, a2, sem_g, sem_s):
    c = lax.axis_index("c")
    s = lax.axis_index("s")
    wid = c * NS + s
    tabs = (t0, t1, t2)
    accs = (a0, a1, a2)
    gbs = (gb0, gb1, gb2)
    sl = pl.ds(s * NPT, NPT)
    _fill(node_v, 0.0, NPT)
    for ch in range(3):
        pltpu.sync_copy(node_v, accs[ch].at[sl])
    for ch in range(3):
        pltpu.sync_copy(g1_hbm.at[pl.ds(ch * NPAD + s * NPT, NPT)], node_v)
        pltpu.sync_copy(node_v, tabs[ch].at[sl])
    plsc.subcore_barrier()

    def chunk(it, _):
        rowbase = wid * TBLK + it * KB
        pltpu.sync_copy(src_hbm.at[pl.ds(rowbase * LANES, CHUNKE)], src_v)
        pltpu.sync_copy(dst_hbm.at[pl.ds(rowbase, KB)], dst_v)
        descs = [pltpu.async_copy(tabs[ch].at[src_v], gbs[ch], sem_g)
                 for ch in range(3)]
        for d in descs:
            d.wait()

        def fire(j, _):
            row = pl.ds(pl.multiple_of(j * LANES, LANES), LANES)
            for ch in range(3):
                pltpu.async_copy(gbs[ch].at[row], accs[ch].at[dst_v.at[j]],
                                 sem_s, add=True)
            return 0

        lax.fori_loop(0, KB, fire, 0)

        def drain(j, _):
            for ch in range(3):
                pltpu.make_async_copy(gbs[ch].at[pl.ds(0, LANES)],
                                      accs[ch].at[dst_v.at[0]], sem_s).wait()
            return 0

        lax.fori_loop(0, KB, drain, 0)
        return 0

    lax.fori_loop(0, NITER, chunk, 0)
    plsc.subcore_barrier()
    for ch in range(3):
        pltpu.sync_copy(accs[ch].at[sl], node_v)
        pltpu.sync_copy(
            node_v, out_hbm.at[pl.ds((c * 3 + ch) * NPAD + s * NPT, NPT)])


@functools.partial(
    pl.kernel,
    out_type=jax.ShapeDtypeStruct((NC * NPAD,), jnp.float32),
    mesh=_mesh,
    scratch_types=[
        pltpu.VMEM((CHUNKE,), jnp.int32),
        pltpu.VMEM((KB, LANES), jnp.int32),
        pltpu.VMEM((CHUNKE,), jnp.float32),
        pltpu.VMEM((NPT,), jnp.float32),
        pltpu.VMEM_SHARED((NPAD,), jnp.float32),
        pltpu.VMEM_SHARED((NPAD,), jnp.float32),
        pltpu.SemaphoreType.DMA,
        pltpu.SemaphoreType.DMA,
    ],
)
def _sc_agg1(src_hbm, dst_hbm, g2_hbm, out_hbm, src_v, dst_v, gb0, node_v,
             tab_s, acc_s, sem_g, sem_s):
    c = lax.axis_index("c")
    s = lax.axis_index("s")
    wid = c * NS + s
    sl = pl.ds(s * NPT, NPT)
    _fill(node_v, 0.0, NPT)
    pltpu.sync_copy(node_v, acc_s.at[sl])
    pltpu.sync_copy(g2_hbm.at[pl.ds(s * NPT, NPT)], node_v)
    pltpu.sync_copy(node_v, tab_s.at[sl])
    plsc.subcore_barrier()

    def chunk(it, _):
        rowbase = wid * TBLK + it * KB
        pltpu.sync_copy(src_hbm.at[pl.ds(rowbase * LANES, CHUNKE)], src_v)
        pltpu.sync_copy(dst_hbm.at[pl.ds(rowbase, KB)], dst_v)
        pltpu.async_copy(tab_s.at[src_v], gb0, sem_g).wait()

        def fire(j, _):
            row = pl.ds(pl.multiple_of(j * LANES, LANES), LANES)
            pltpu.async_copy(gb0.at[row],
                             acc_s.at[dst_v.at[j]], sem_s, add=True)
            return 0

        lax.fori_loop(0, KB, fire, 0)

        def drain(j, _):
            pltpu.make_async_copy(gb0.at[pl.ds(0, LANES)],
                                  acc_s.at[dst_v.at[0]], sem_s).wait()
            return 0

        lax.fori_loop(0, KB, drain, 0)
        return 0

    lax.fori_loop(0, NITER, chunk, 0)
    plsc.subcore_barrier()
    pltpu.sync_copy(acc_s.at[sl], node_v)
    pltpu.sync_copy(node_v, out_hbm.at[pl.ds(c * NPAD + s * NPT, NPT)])


def _tc_norm_body(degp_ref, xq_ref, dinv_ref, g1_ref):
    deg = degp_ref[0] + degp_ref[1] + 1.0
    dinv = deg ** -0.5
    dinv_ref[...] = dinv
    g1_ref[...] = xq_ref[...] * dinv[None]


def _tc_dense_body(aggp_ref, g1_ref, dinv_ref, w1_ref, b1_ref, w2_ref,
                   g2_ref):
    dinv = dinv_ref[...]
    xa = [(aggp_ref[0, ch] + aggp_ref[1, ch] + g1_ref[ch]) * dinv
          for ch in range(3)]
    s = jnp.zeros((ROWS, LANES), jnp.float32)
    for o in range(16):
        h = (xa[0] * w1_ref[0, o] + xa[1] * w1_ref[1, o]
             + xa[2] * w1_ref[2, o] + b1_ref[o])
        s = s + jnp.maximum(h, 0.0) * w2_ref[o, 0]
    g2_ref[...] = s * dinv


def _tc_final_body(a2p_ref, g2_ref, dinv_ref, b2_ref, out_ref):
    out_ref[...] = ((a2p_ref[0] + a2p_ref[1] + g2_ref[...])
                    * dinv_ref[...] + b2_ref[0])


def kernel(x, edge_index, W1, b1, W2, b2):
    f32 = jnp.float32
    ei = edge_index.astype(jnp.int32)
    npad_e = EPAD - N_EDGES_C
    pad_idx = (N_NODES_C
               + jnp.arange(npad_e, dtype=jnp.int32) % (NPAD - N_NODES_C))
    src1 = jnp.concatenate([ei[0], pad_idx])
    dst2 = jnp.concatenate([ei[1], pad_idx]).reshape(EPAD // LANES, LANES)

    xp = jnp.pad(x.astype(f32), ((0, NPAD - N_NODES_C), (0, 0)))
    xq = xp.T.reshape(3, ROWS, LANES)

    degp = _sc_degree(dst2)

    dinv2, g1 = pl.pallas_call(
        _tc_norm_body,
        out_shape=[
            jax.ShapeDtypeStruct((ROWS, LANES), f32),
            jax.ShapeDtypeStruct((3, ROWS, LANES), f32),
        ],
    )(degp.reshape(NC, ROWS, LANES), xq)

    aggp = _sc_agg3(src1, dst2, g1.reshape(3 * NPAD))

    g2 = pl.pallas_call(
        _tc_dense_body,
        in_specs=[
            pl.BlockSpec(memory_space=pltpu.VMEM),
            pl.BlockSpec(memory_space=pltpu.VMEM),
            pl.BlockSpec(memory_space=pltpu.VMEM),
            pl.BlockSpec(memory_space=pltpu.SMEM),
            pl.BlockSpec(memory_space=pltpu.SMEM),
            pl.BlockSpec(memory_space=pltpu.SMEM),
        ],
        out_shape=jax.ShapeDtypeStruct((ROWS, LANES), f32),
    )(aggp.reshape(NC, 3, ROWS, LANES), g1, dinv2,
      W1.astype(f32), b1.astype(f32), W2.astype(f32))

    a2p = _sc_agg1(src1, dst2, g2.reshape(NPAD))

    out2 = pl.pallas_call(
        _tc_final_body,
        in_specs=[
            pl.BlockSpec(memory_space=pltpu.VMEM),
            pl.BlockSpec(memory_space=pltpu.VMEM),
            pl.BlockSpec(memory_space=pltpu.VMEM),
            pl.BlockSpec(memory_space=pltpu.SMEM),
        ],
        out_shape=jax.ShapeDtypeStruct((ROWS, LANES), f32),
    )(a2p.reshape(NC, ROWS, LANES), g2, dinv2, b2.astype(f32))

    return out2.reshape(NPAD)[:N_NODES_C]
